# baseline (speedup 1.0000x reference)
import functools

import jax
import jax.numpy as jnp
from jax.experimental import pallas as pl
from jax.experimental.pallas import tpu as pltpu

BLK = 4096


def _lower_bound(skeys_ref, u, nsteps, target):
    def step(_, lohi):
        lo, hi = lohi
        mid = jnp.minimum((lo + hi) // 2, u - 1)
        pred = jnp.logical_and(lo < hi, skeys_ref[mid] < target)
        lo2 = jnp.where(pred, mid + 1, lo)
        hi2 = jnp.where(jnp.logical_and(lo < hi, jnp.logical_not(pred)),
                        mid, hi)
        return lo2, hi2

    lo, _ = jax.lax.fori_loop(0, nsteps, step, (0, u))
    return lo


def _body(x_ref, w_ref, b_ref, dv_ref, skeys_ref, o_ref, *, shift):
    k = pl.program_id(0)
    u = skeys_ref.shape[0]
    nsteps = max(u.bit_length() + 1, 2)

    def fix(s, carry):
        e = skeys_ref[s]
        local = (e >> shift) - k * BLK
        src = e & ((1 << shift) - 1)
        w_ref[pl.ds(local, 1), :] = dv_ref[pl.ds(src, 1), :]
        return carry

    s0 = _lower_bound(skeys_ref, u, nsteps, (k * BLK) << shift)
    s1 = _lower_bound(skeys_ref, u, nsteps, ((k + 1) * BLK) << shift)
    jax.lax.fori_loop(s0, s1, fix, 0)

    acc = jax.lax.dot_general(
        x_ref[...], w_ref[...], (((1,), (1,)), ((), ())),
        preferred_element_type=jnp.float32)
    o_ref[...] = jnp.maximum(acc + b_ref[...], 0.0)


def kernel(x, dict_idx, dict_val, W, b):
    L, F = W.shape
    B = x.shape[0]
    U = dict_idx.shape[0]
    nblk = L // BLK
    shift = max((U - 1).bit_length(), 1)

    keys = ((dict_idx.astype(jnp.int32) << shift)
            | jnp.arange(U, dtype=jnp.int32))
    skeys = jax.lax.sort(keys)

    b2 = b.reshape(1, L)

    out = pl.pallas_call(
        functools.partial(_body, shift=shift),
        grid=(nblk,),
        in_specs=[
            pl.BlockSpec((B, F), lambda k: (0, 0)),
            pl.BlockSpec((BLK, F), lambda k: (k, 0)),
            pl.BlockSpec((1, BLK), lambda k: (0, k)),
            pl.BlockSpec((U, F), lambda k: (0, 0)),
            pl.BlockSpec(memory_space=pltpu.SMEM),
        ],
        out_specs=pl.BlockSpec((B, BLK), lambda k: (0, k)),
        out_shape=jax.ShapeDtypeStruct((B, L), jnp.float32),
    )(x, W, b2, dict_val, skeys)
    return out
